# Initial kernel scaffold; baseline (speedup 1.0000x reference)
import jax
import jax.numpy as jnp
from jax.experimental import pallas as pl


def kernel(x, edge_index, W1, a_src1, a_dst1, b1, W2, a_src2, a_dst2, b2, W3, a_src3, a_dst3, b3):
    raise NotImplementedError("write your pallas kernel here")


# speedup vs baseline: 1.0150x; 1.0150x over previous
import jax
import jax.numpy as jnp
from jax.experimental import pallas as pl

N_NODES = 10000
HEADS = 8
HDIM = 64


def _matmul_kernel(x_ref, w_ref, o_ref):
    o_ref[...] = jnp.dot(x_ref[...], w_ref[...], preferred_element_type=jnp.float32)


def _matmul(x, w, block_rows=1000):
    n, k = x.shape
    m = w.shape[1]
    grid = (n // block_rows,)
    return pl.pallas_call(
        _matmul_kernel,
        grid=grid,
        in_specs=[
            pl.BlockSpec((block_rows, k), lambda i: (i, 0)),
            pl.BlockSpec((k, m), lambda i: (0, 0)),
        ],
        out_specs=pl.BlockSpec((block_rows, m), lambda i: (i, 0)),
        out_shape=jax.ShapeDtypeStruct((n, m), jnp.float32),
    )(x, w)


def _gat_layer(x, src, dst, W, a_src, a_dst, bias, heads, out_ch, concat):
    n = x.shape[0]
    h = _matmul(x, W).reshape(n, heads, out_ch)
    alpha_src = (h * a_src).sum(-1)
    alpha_dst = (h * a_dst).sum(-1)
    e = alpha_src[src] + alpha_dst[dst]
    e = jax.nn.leaky_relu(e, 0.2)
    emax = jax.ops.segment_max(e, dst, num_segments=n)
    emax = jnp.where(jnp.isfinite(emax), emax, 0.0)
    ee = jnp.exp(e - emax[dst])
    denom = jax.ops.segment_sum(ee, dst, num_segments=n)
    alpha = ee / (denom[dst] + 1e-16)
    msg = h[src] * alpha[..., None]
    out = jax.ops.segment_sum(msg, dst, num_segments=n)
    if concat:
        out = out.reshape(n, heads * out_ch)
    else:
        out = out.mean(axis=1)
    return out + bias


def kernel(x, edge_index, W1, a_src1, a_dst1, b1, W2, a_src2, a_dst2, b2, W3, a_src3, a_dst3, b3):
    n = x.shape[0]
    loop = jnp.arange(n)
    src = jnp.concatenate([edge_index[0], loop])
    dst = jnp.concatenate([edge_index[1], loop])
    h = _gat_layer(x.astype(jnp.float32), src, dst, W1, a_src1, a_dst1, b1, HEADS, HDIM // HEADS, True)
    h = jax.nn.relu(h)
    h = _gat_layer(h, src, dst, W2, a_src2, a_dst2, b2, HEADS, HDIM // HEADS, True)
    h = jax.nn.relu(h)
    h = _gat_layer(h, src, dst, W3, a_src3, a_dst3, b3, 1, 242, False)
    h = jax.nn.relu(h)
    h = h.reshape(n, -1, 2)
    return jax.nn.softmax(h, axis=-1)


# speedup vs baseline: 32.6108x; 32.1297x over previous
import functools

import jax
import jax.numpy as jnp
from jax import lax
from jax.experimental import pallas as pl
from jax.experimental.pallas import tpu as pltpu
from jax.experimental.pallas import tpu_sc as plsc

N = 10000
N_PAD = 10240
NC = 2
NS = 16
CHUNK = 128
CPT = 81
E_PAD = NC * NS * CPT * CHUNK
RPS = N_PAD // NS
NEGH = -5e29
F32 = jnp.float32


def _sc_mesh():
    return plsc.VectorSubcoreMesh(
        core_axis_name="c", subcore_axis_name="s", num_cores=NC, num_subcores=NS)


def _pass1_body(src_hbm, dst_hbm, asrc_hbm, adst_hbm, m16_hbm, z16_hbm,
                ee_hbm, dpart_hbm,
                src_v, dst_v, rs_v, rd_v, ee_v, m16_v, den_sp, sem):
    c = lax.axis_index("c")
    s = lax.axis_index("s")
    wid = c * NS + s
    pltpu.sync_copy(z16_hbm.at[pl.ds(s * RPS, RPS)],
                    den_sp.at[pl.ds(s * RPS, RPS)])
    pltpu.sync_copy(m16_hbm, m16_v)
    pltpu.sync_copy(src_hbm.at[wid], src_v)
    pltpu.sync_copy(dst_hbm.at[wid], dst_v)
    plsc.subcore_barrier()
    m16 = m16_v[...]

    def chunk_body(j, carry):
        idx_s = src_v.at[j]
        idx_d = dst_v.at[j]
        pltpu.async_copy(asrc_hbm.at[idx_s], rs_v, sem).wait()
        pltpu.async_copy(adst_hbm.at[idx_d], rd_v, sem).wait()

        def edge_body(k, carry2):
            e = rs_v[k] + rd_v[k]
            e = jnp.where(e > 0, e, 0.2 * e)
            ee_v[k] = jnp.exp(e - m16)
            return carry2

        lax.fori_loop(0, CHUNK, edge_body, 0, unroll=4)
        pltpu.sync_copy(ee_v, den_sp.at[idx_d], add=True)
        pltpu.sync_copy(ee_v, ee_hbm.at[wid, j])
        return carry

    lax.fori_loop(0, CPT, chunk_body, 0)
    plsc.subcore_barrier()
    pltpu.sync_copy(den_sp.at[pl.ds(s * RPS, RPS)],
                    dpart_hbm.at[c, pl.ds(s * RPS, RPS)])


def _make_pass1():
    return functools.partial(
        pl.kernel,
        out_type=[
            jax.ShapeDtypeStruct((NC * NS, CPT, CHUNK, 16), F32),
            jax.ShapeDtypeStruct((NC, N_PAD, 16), F32),
        ],
        mesh=_sc_mesh(),
        compiler_params=pltpu.CompilerParams(use_tc_tiling_on_sc=False, needs_layout_passes=False),
        scratch_types=[
            pltpu.VMEM((CPT, CHUNK), jnp.int32),
            pltpu.VMEM((CPT, CHUNK), jnp.int32),
            pltpu.VMEM((CHUNK, 16), F32),
            pltpu.VMEM((CHUNK, 16), F32),
            pltpu.VMEM((CHUNK, 16), F32),
            pltpu.VMEM((16,), F32),
            pltpu.VMEM_SHARED((N_PAD, 16), F32),
            pltpu.SemaphoreType.DMA,
        ],
    )(_pass1_body)


def _make_pass2(heads8):
    def body(src_hbm, dst_hbm, ee_hbm, den_hbm, h_hbm, z64_hbm,
             opart_hbm,
             src_v, dst_v, ee_v, dn_v, hs_v, al_v, msg_v, out_sp, sem):
        c = lax.axis_index("c")
        s = lax.axis_index("s")
        wid = c * NS + s
        pltpu.sync_copy(z64_hbm.at[pl.ds(s * RPS, RPS)],
                        out_sp.at[pl.ds(s * RPS, RPS)])
        pltpu.sync_copy(src_hbm.at[wid], src_v)
        pltpu.sync_copy(dst_hbm.at[wid], dst_v)
        plsc.subcore_barrier()
        iota16 = lax.broadcasted_iota(jnp.int32, (16,), 0)

        def chunk_body(j, carry):
            idx_s = src_v.at[j]
            idx_d = dst_v.at[j]
            pltpu.async_copy(den_hbm.at[idx_d], dn_v, sem).wait()
            pltpu.async_copy(h_hbm.at[idx_s], hs_v, sem).wait()
            pltpu.sync_copy(ee_hbm.at[wid, j], ee_v)

            def edge_body(k, carry2):
                al_v[k] = ee_v[k] / (dn_v[k] + 1e-16)
                ksplat = jnp.full((16,), 0, jnp.int32) + k
                for v in range(4):
                    if heads8:
                        pat = 2 * v + jnp.where(iota16 >= 8, 1, 0)
                    else:
                        pat = iota16 * 0
                    av = plsc.load_gather(al_v, [ksplat, pat])
                    msg_v[k, pl.ds(16 * v, 16)] = hs_v[k, pl.ds(16 * v, 16)] * av
                return carry2

            lax.fori_loop(0, CHUNK, edge_body, 0, unroll=2)
            pltpu.sync_copy(msg_v, out_sp.at[idx_d], add=True)
            return carry

        lax.fori_loop(0, CPT, chunk_body, 0)
        plsc.subcore_barrier()
        pltpu.sync_copy(out_sp.at[pl.ds(s * RPS, RPS)],
                        opart_hbm.at[c, pl.ds(s * RPS, RPS)])

    return functools.partial(
        pl.kernel,
        out_type=[jax.ShapeDtypeStruct((NC, N_PAD, 64), F32)],
        mesh=_sc_mesh(),
        compiler_params=pltpu.CompilerParams(use_tc_tiling_on_sc=False, needs_layout_passes=False),
        scratch_types=[
            pltpu.VMEM((CPT, CHUNK), jnp.int32),
            pltpu.VMEM((CPT, CHUNK), jnp.int32),
            pltpu.VMEM((CHUNK, 16), F32),
            pltpu.VMEM((CHUNK, 16), F32),
            pltpu.VMEM((CHUNK, 64), F32),
            pltpu.VMEM((CHUNK, 16), F32),
            pltpu.VMEM((CHUNK, 64), F32),
            pltpu.VMEM_SHARED((N_PAD, 64), F32),
            pltpu.SemaphoreType.DMA,
        ],
    )(body)


_BLK = 512
_NBLK = N_PAD // _BLK


def _tables1_body(x_ref, w_ref, bs_ref, bd_ref, pv_ref, h_ref, as_ref, ad_ref):
    h = jnp.dot(x_ref[...], w_ref[...], preferred_element_type=F32)
    h_ref[...] = h
    as_ref[...] = jnp.dot(h, bs_ref[...], preferred_element_type=F32) + pv_ref[...]
    ad_ref[...] = jnp.dot(h, bd_ref[...], preferred_element_type=F32) + pv_ref[...]


def _tables_next_body(p0_ref, p1_ref, b_ref, w_ref, bs_ref, bd_ref, pv_ref,
                      h_ref, as_ref, ad_ref):
    xx = jax.nn.relu(p0_ref[...] + p1_ref[...] + b_ref[...])
    h = jnp.dot(xx, w_ref[...], preferred_element_type=F32)
    h_ref[...] = h
    as_ref[...] = jnp.dot(h, bs_ref[...], preferred_element_type=F32) + pv_ref[...]
    ad_ref[...] = jnp.dot(h, bd_ref[...], preferred_element_type=F32) + pv_ref[...]


def _tables3_body(p0_ref, p1_ref, b_ref, w_ref, as3_ref, ad3_ref, pv_ref,
                  x_ref, as_ref, ad_ref):
    xx = jax.nn.relu(p0_ref[...] + p1_ref[...] + b_ref[...])
    x_ref[...] = xx
    hw = jnp.dot(xx, w_ref[...], preferred_element_type=F32)
    as_ref[...] = jnp.dot(hw, as3_ref[...], preferred_element_type=F32) + pv_ref[...]
    ad_ref[...] = jnp.dot(hw, ad3_ref[...], preferred_element_type=F32) + pv_ref[...]


def _m16_body(as_ref, ad_ref, o_ref):
    m = jnp.max(as_ref[...], axis=0) + jnp.max(ad_ref[...], axis=0)
    m = jnp.where(m > 0, m, 0.2 * m)
    o_ref[...] = jnp.broadcast_to(m[None, :], (8, 16))


def _addden_body(dp_ref, o_ref):
    o_ref[...] = dp_ref[0] + dp_ref[1]


def _final_body(p0_ref, p1_ref, b3e_ref, b3o_ref, w3e_ref, w3o_ref,
                s0_ref, s1_ref):
    agg = p0_ref[...] + p1_ref[...]
    z0 = jax.nn.relu(jnp.dot(agg, w3e_ref[...], preferred_element_type=F32)
                     + b3e_ref[...])
    z1 = jax.nn.relu(jnp.dot(agg, w3o_ref[...], preferred_element_type=F32)
                     + b3o_ref[...])
    m = jnp.maximum(z0, z1)
    e0 = jnp.exp(z0 - m)
    e1 = jnp.exp(z1 - m)
    t = e0 + e1
    s0_ref[...] = e0 / t
    s1_ref[...] = e1 / t


def _row_spec(cols):
    return pl.BlockSpec((_BLK, cols), lambda i: (i, 0))


def _full_spec(shape):
    return pl.BlockSpec(shape, lambda i: tuple(0 for _ in shape))


def _tables1(xp, W1, Bs, Bd, pv):
    return pl.pallas_call(
        _tables1_body,
        grid=(_NBLK,),
        in_specs=[_row_spec(128), _full_spec((128, 64)), _full_spec((64, 16)),
                  _full_spec((64, 16)), _full_spec((1, 16))],
        out_specs=[_row_spec(64), _row_spec(16), _row_spec(16)],
        out_shape=[jax.ShapeDtypeStruct((N_PAD, 64), F32),
                   jax.ShapeDtypeStruct((N_PAD, 16), F32),
                   jax.ShapeDtypeStruct((N_PAD, 16), F32)],
    )(xp, W1, Bs, Bd, pv)


def _tables_next(p0, p1, b, W, Bs, Bd, pv):
    return pl.pallas_call(
        _tables_next_body,
        grid=(_NBLK,),
        in_specs=[_row_spec(64), _row_spec(64), _full_spec((1, 64)),
                  _full_spec((64, 64)), _full_spec((64, 16)),
                  _full_spec((64, 16)), _full_spec((1, 16))],
        out_specs=[_row_spec(64), _row_spec(16), _row_spec(16)],
        out_shape=[jax.ShapeDtypeStruct((N_PAD, 64), F32),
                   jax.ShapeDtypeStruct((N_PAD, 16), F32),
                   jax.ShapeDtypeStruct((N_PAD, 16), F32)],
    )(p0, p1, b, W, Bs, Bd, pv)


def _tables3(p0, p1, b, W3, A_s, A_d, pv):
    return pl.pallas_call(
        _tables3_body,
        grid=(_NBLK,),
        in_specs=[_row_spec(64), _row_spec(64), _full_spec((1, 64)),
                  _full_spec((64, 242)), _full_spec((242, 16)),
                  _full_spec((242, 16)), _full_spec((1, 16))],
        out_specs=[_row_spec(64), _row_spec(16), _row_spec(16)],
        out_shape=[jax.ShapeDtypeStruct((N_PAD, 64), F32),
                   jax.ShapeDtypeStruct((N_PAD, 16), F32),
                   jax.ShapeDtypeStruct((N_PAD, 16), F32)],
    )(p0, p1, b, W3, A_s, A_d, pv)


def _m16(asrc, adst):
    out = pl.pallas_call(
        _m16_body,
        out_shape=jax.ShapeDtypeStruct((8, 16), F32),
    )(asrc, adst)
    return out[0]


def _addden(dp):
    return pl.pallas_call(
        _addden_body,
        out_shape=jax.ShapeDtypeStruct((N_PAD, 16), F32),
    )(dp)


def _final(p0, p1, b3e, b3o, W3e, W3o):
    return pl.pallas_call(
        _final_body,
        grid=(_NBLK,),
        in_specs=[_row_spec(64), _row_spec(64), _full_spec((1, 121)),
                  _full_spec((1, 121)), _full_spec((64, 121)),
                  _full_spec((64, 121))],
        out_specs=[_row_spec(121), _row_spec(121)],
        out_shape=[jax.ShapeDtypeStruct((N_PAD, 121), F32),
                   jax.ShapeDtypeStruct((N_PAD, 121), F32)],
    )(p0, p1, b3e, b3o, W3e, W3o)


def _blockdiag(a):
    eye8 = jnp.eye(8, dtype=F32)
    B = (a.astype(F32)[:, :, None] * eye8[:, None, :]).reshape(64, 8)
    return jnp.pad(B, ((0, 0), (0, 8)))


def kernel(x, edge_index, W1, a_src1, a_dst1, b1, W2, a_src2, a_dst2, b2,
           W3, a_src3, a_dst3, b3):
    x = x.astype(F32)
    loop = jnp.arange(N, dtype=jnp.int32)
    src = jnp.concatenate([edge_index[0].astype(jnp.int32), loop])
    dst = jnp.concatenate([edge_index[1].astype(jnp.int32), loop])
    pad_e = E_PAD - src.shape[0]
    src = jnp.concatenate([src, jnp.full((pad_e,), N, jnp.int32)])
    dst = jnp.concatenate([dst, jnp.full((pad_e,), N, jnp.int32)])
    src = src.reshape(NC * NS, CPT, CHUNK)
    dst = dst.reshape(NC * NS, CPT, CHUNK)

    xp = jnp.pad(x, ((0, N_PAD - N), (0, 0)))
    Bs1, Bd1 = _blockdiag(a_src1), _blockdiag(a_dst1)
    Bs2, Bd2 = _blockdiag(a_src2), _blockdiag(a_dst2)
    A_s = jnp.pad(jnp.tile(a_src3.astype(F32).reshape(242, 1), (1, 8)),
                  ((0, 0), (0, 8)))
    A_d = jnp.pad(jnp.tile(a_dst3.astype(F32).reshape(242, 1), (1, 8)),
                  ((0, 0), (0, 8)))
    pv = jnp.concatenate([jnp.zeros((8,), F32),
                          jnp.full((8,), NEGH, F32)]).reshape(1, 16)
    z16 = jnp.zeros((N_PAD, 16), F32)
    z64 = jnp.zeros((N_PAD, 64), F32)
    W3e = W3.astype(F32)[:, 0::2]
    W3o = W3.astype(F32)[:, 1::2]
    b3e = b3.astype(F32)[0::2].reshape(1, 121)
    b3o = b3.astype(F32)[1::2].reshape(1, 121)
    b1r = b1.astype(F32).reshape(1, 64)
    b2r = b2.astype(F32).reshape(1, 64)

    pass1 = _make_pass1()
    pass2_h8 = _make_pass2(True)
    pass2_h1 = _make_pass2(False)

    h1, as1, ad1 = _tables1(xp, W1.astype(F32), Bs1, Bd1, pv)
    m1 = _m16(as1, ad1)
    ee1, dp1 = pass1(src, dst, as1, ad1, m1, z16)
    den1 = _addden(dp1)
    op1, = pass2_h8(src, dst, ee1, den1, h1, z64)
    h2, as2, ad2 = _tables_next(op1[0], op1[1], b1r, W2.astype(F32),
                                Bs2, Bd2, pv)
    m2 = _m16(as2, ad2)
    ee2, dp2 = pass1(src, dst, as2, ad2, m2, z16)
    den2 = _addden(dp2)
    op2, = pass2_h8(src, dst, ee2, den2, h2, z64)
    x3, as3, ad3 = _tables3(op2[0], op2[1], b2r, W3.astype(F32), A_s, A_d, pv)
    m3 = _m16(as3, ad3)
    ee3, dp3 = pass1(src, dst, as3, ad3, m3, z16)
    den3 = _addden(dp3)
    op3, = pass2_h1(src, dst, ee3, den3, x3, z64)
    s0, s1 = _final(op3[0], op3[1], b3e, b3o, W3e, W3o)
    return jnp.stack([s0[:N], s1[:N]], axis=-1)


# speedup vs baseline: 39.9836x; 1.2261x over previous
import functools

import jax
import jax.numpy as jnp
from jax import lax
from jax.experimental import pallas as pl
from jax.experimental.pallas import tpu as pltpu
from jax.experimental.pallas import tpu_sc as plsc

N = 10000
N_PAD = 10240
NC = 2
NS = 16
CHUNK = 128
CPT = 81
E_PAD = NC * NS * CPT * CHUNK
RPS = N_PAD // NS
NEGH = -5e29
F32 = jnp.float32


def _sc_mesh():
    return plsc.VectorSubcoreMesh(
        core_axis_name="c", subcore_axis_name="s", num_cores=NC, num_subcores=NS)


def _pass1_body(src_hbm, dst_hbm, asrc_hbm, adst_hbm, m16_hbm, z16_hbm,
                ee_hbm, dpart_hbm,
                src_v, dst_v, rs_v, rd_v, ee_v, m16_v, den_sp, sem):
    c = lax.axis_index("c")
    s = lax.axis_index("s")
    wid = c * NS + s
    pltpu.sync_copy(z16_hbm.at[pl.ds(s * RPS, RPS)],
                    den_sp.at[pl.ds(s * RPS, RPS)])
    pltpu.sync_copy(m16_hbm, m16_v)
    pltpu.sync_copy(src_hbm.at[wid], src_v)
    pltpu.sync_copy(dst_hbm.at[wid], dst_v)
    plsc.subcore_barrier()
    m16 = m16_v[...]

    def chunk_body(j, carry):
        idx_s = src_v.at[j]
        idx_d = dst_v.at[j]
        pltpu.async_copy(asrc_hbm.at[idx_s], rs_v, sem).wait()
        pltpu.async_copy(adst_hbm.at[idx_d], rd_v, sem).wait()

        def edge_body(k, carry2):
            e = rs_v[k] + rd_v[k]
            e = jnp.where(e > 0, e, 0.2 * e)
            ee_v[k] = jnp.exp(e - m16)
            return carry2

        lax.fori_loop(0, CHUNK, edge_body, 0, unroll=4)
        pltpu.sync_copy(ee_v, den_sp.at[idx_d], add=True)
        pltpu.sync_copy(ee_v, ee_hbm.at[wid, j])
        return carry

    lax.fori_loop(0, CPT, chunk_body, 0)
    plsc.subcore_barrier()
    pltpu.sync_copy(den_sp.at[pl.ds(s * RPS, RPS)],
                    dpart_hbm.at[c, pl.ds(s * RPS, RPS)])


def _make_pass1():
    return functools.partial(
        pl.kernel,
        out_type=[
            jax.ShapeDtypeStruct((NC * NS, CPT, CHUNK, 16), F32),
            jax.ShapeDtypeStruct((NC, N_PAD, 16), F32),
        ],
        mesh=_sc_mesh(),
        compiler_params=pltpu.CompilerParams(use_tc_tiling_on_sc=False, needs_layout_passes=False),
        scratch_types=[
            pltpu.VMEM((CPT, CHUNK), jnp.int32),
            pltpu.VMEM((CPT, CHUNK), jnp.int32),
            pltpu.VMEM((CHUNK, 16), F32),
            pltpu.VMEM((CHUNK, 16), F32),
            pltpu.VMEM((CHUNK, 16), F32),
            pltpu.VMEM((16,), F32),
            pltpu.VMEM_SHARED((N_PAD, 16), F32),
            pltpu.SemaphoreType.DMA,
        ],
    )(_pass1_body)


def _make_pass2(heads8):
    def body(src_hbm, dst_hbm, ee_hbm, h_hbm, z64_hbm,
             opart_hbm,
             src_v, dst_v, ee_v, hs_v, msg_v, out_sp, sem):
        c = lax.axis_index("c")
        s = lax.axis_index("s")
        wid = c * NS + s
        pltpu.sync_copy(z64_hbm.at[pl.ds(s * RPS, RPS)],
                        out_sp.at[pl.ds(s * RPS, RPS)])
        pltpu.sync_copy(src_hbm.at[wid], src_v)
        pltpu.sync_copy(dst_hbm.at[wid], dst_v)
        plsc.subcore_barrier()
        iota16 = lax.broadcasted_iota(jnp.int32, (16,), 0)

        def chunk_body(j, carry):
            idx_s = src_v.at[j]
            idx_d = dst_v.at[j]
            pltpu.async_copy(h_hbm.at[idx_s], hs_v, sem).wait()
            pltpu.sync_copy(ee_hbm.at[wid, j], ee_v)

            def edge_body(k, carry2):
                ksplat = jnp.full((16,), 0, jnp.int32) + k
                for v in range(4):
                    if heads8:
                        pat = 2 * v + jnp.where(iota16 >= 8, 1, 0)
                    else:
                        pat = iota16 * 0
                    av = plsc.load_gather(ee_v, [ksplat, pat])
                    msg_v[k, pl.ds(16 * v, 16)] = hs_v[k, pl.ds(16 * v, 16)] * av
                return carry2

            lax.fori_loop(0, CHUNK, edge_body, 0, unroll=2)
            pltpu.sync_copy(msg_v, out_sp.at[idx_d], add=True)
            return carry

        lax.fori_loop(0, CPT, chunk_body, 0)
        plsc.subcore_barrier()
        pltpu.sync_copy(out_sp.at[pl.ds(s * RPS, RPS)],
                        opart_hbm.at[c, pl.ds(s * RPS, RPS)])

    return functools.partial(
        pl.kernel,
        out_type=[jax.ShapeDtypeStruct((NC, N_PAD, 64), F32)],
        mesh=_sc_mesh(),
        compiler_params=pltpu.CompilerParams(use_tc_tiling_on_sc=False, needs_layout_passes=False),
        scratch_types=[
            pltpu.VMEM((CPT, CHUNK), jnp.int32),
            pltpu.VMEM((CPT, CHUNK), jnp.int32),
            pltpu.VMEM((CHUNK, 16), F32),
            pltpu.VMEM((CHUNK, 64), F32),
            pltpu.VMEM((CHUNK, 64), F32),
            pltpu.VMEM_SHARED((N_PAD, 64), F32),
            pltpu.SemaphoreType.DMA,
        ],
    )(body)


_BLK = 512
_NBLK = N_PAD // _BLK


def _tables1_body(x_ref, w_ref, bs_ref, bd_ref, pv_ref, h_ref, as_ref, ad_ref):
    h = jnp.dot(x_ref[...], w_ref[...], preferred_element_type=F32)
    h_ref[...] = h
    as_ref[...] = jnp.dot(h, bs_ref[...], preferred_element_type=F32) + pv_ref[...]
    ad_ref[...] = jnp.dot(h, bd_ref[...], preferred_element_type=F32) + pv_ref[...]


def _tables_next_body(p0_ref, p1_ref, d0_ref, d1_ref, ex_ref, b_ref, w_ref,
                      bs_ref, bd_ref, pv_ref, h_ref, as_ref, ad_ref):
    dexp = jnp.dot(d0_ref[...] + d1_ref[...], ex_ref[...],
                   preferred_element_type=F32) + 1e-16
    xx = jax.nn.relu((p0_ref[...] + p1_ref[...]) / dexp + b_ref[...])
    h = jnp.dot(xx, w_ref[...], preferred_element_type=F32)
    h_ref[...] = h
    as_ref[...] = jnp.dot(h, bs_ref[...], preferred_element_type=F32) + pv_ref[...]
    ad_ref[...] = jnp.dot(h, bd_ref[...], preferred_element_type=F32) + pv_ref[...]


def _tables3_body(p0_ref, p1_ref, d0_ref, d1_ref, ex_ref, b_ref, w_ref,
                  as3_ref, ad3_ref, pv_ref, x_ref, as_ref, ad_ref):
    dexp = jnp.dot(d0_ref[...] + d1_ref[...], ex_ref[...],
                   preferred_element_type=F32) + 1e-16
    xx = jax.nn.relu((p0_ref[...] + p1_ref[...]) / dexp + b_ref[...])
    x_ref[...] = xx
    hw = jnp.dot(xx, w_ref[...], preferred_element_type=F32)
    as_ref[...] = jnp.dot(hw, as3_ref[...], preferred_element_type=F32) + pv_ref[...]
    ad_ref[...] = jnp.dot(hw, ad3_ref[...], preferred_element_type=F32) + pv_ref[...]


def _m16_body(as_ref, ad_ref, o_ref):
    m = jnp.max(as_ref[...], axis=0) + jnp.max(ad_ref[...], axis=0)
    m = jnp.where(m > 0, m, 0.2 * m)
    o_ref[...] = jnp.broadcast_to(m[None, :], (8, 16))


def _final_body(p0_ref, p1_ref, d0_ref, d1_ref, ex_ref, b3e_ref, b3o_ref,
                w3e_ref, w3o_ref, s0_ref, s1_ref):
    dexp = jnp.dot(d0_ref[...] + d1_ref[...], ex_ref[...],
                   preferred_element_type=F32) + 1e-16
    agg = (p0_ref[...] + p1_ref[...]) / dexp
    z0 = jax.nn.relu(jnp.dot(agg, w3e_ref[...], preferred_element_type=F32)
                     + b3e_ref[...])
    z1 = jax.nn.relu(jnp.dot(agg, w3o_ref[...], preferred_element_type=F32)
                     + b3o_ref[...])
    m = jnp.maximum(z0, z1)
    e0 = jnp.exp(z0 - m)
    e1 = jnp.exp(z1 - m)
    t = e0 + e1
    s0_ref[...] = e0 / t
    s1_ref[...] = e1 / t


def _row_spec(cols):
    return pl.BlockSpec((_BLK, cols), lambda i: (i, 0))


def _full_spec(shape):
    return pl.BlockSpec(shape, lambda i: tuple(0 for _ in shape))


def _tables1(xp, W1, Bs, Bd, pv):
    return pl.pallas_call(
        _tables1_body,
        grid=(_NBLK,),
        in_specs=[_row_spec(128), _full_spec((128, 64)), _full_spec((64, 16)),
                  _full_spec((64, 16)), _full_spec((1, 16))],
        out_specs=[_row_spec(64), _row_spec(16), _row_spec(16)],
        out_shape=[jax.ShapeDtypeStruct((N_PAD, 64), F32),
                   jax.ShapeDtypeStruct((N_PAD, 16), F32),
                   jax.ShapeDtypeStruct((N_PAD, 16), F32)],
    )(xp, W1, Bs, Bd, pv)


def _tables_next(p0, p1, d0, d1, ex, b, W, Bs, Bd, pv):
    return pl.pallas_call(
        _tables_next_body,
        grid=(_NBLK,),
        in_specs=[_row_spec(64), _row_spec(64), _row_spec(16), _row_spec(16),
                  _full_spec((16, 64)), _full_spec((1, 64)),
                  _full_spec((64, 64)), _full_spec((64, 16)),
                  _full_spec((64, 16)), _full_spec((1, 16))],
        out_specs=[_row_spec(64), _row_spec(16), _row_spec(16)],
        out_shape=[jax.ShapeDtypeStruct((N_PAD, 64), F32),
                   jax.ShapeDtypeStruct((N_PAD, 16), F32),
                   jax.ShapeDtypeStruct((N_PAD, 16), F32)],
    )(p0, p1, d0, d1, ex, b, W, Bs, Bd, pv)


def _tables3(p0, p1, d0, d1, ex, b, W3, A_s, A_d, pv):
    return pl.pallas_call(
        _tables3_body,
        grid=(_NBLK,),
        in_specs=[_row_spec(64), _row_spec(64), _row_spec(16), _row_spec(16),
                  _full_spec((16, 64)), _full_spec((1, 64)),
                  _full_spec((64, 242)), _full_spec((242, 16)),
                  _full_spec((242, 16)), _full_spec((1, 16))],
        out_specs=[_row_spec(64), _row_spec(16), _row_spec(16)],
        out_shape=[jax.ShapeDtypeStruct((N_PAD, 64), F32),
                   jax.ShapeDtypeStruct((N_PAD, 16), F32),
                   jax.ShapeDtypeStruct((N_PAD, 16), F32)],
    )(p0, p1, d0, d1, ex, b, W3, A_s, A_d, pv)


def _m16(asrc, adst):
    out = pl.pallas_call(
        _m16_body,
        out_shape=jax.ShapeDtypeStruct((8, 16), F32),
    )(asrc, adst)
    return out[0]


def _final(p0, p1, d0, d1, ex, b3e, b3o, W3e, W3o):
    return pl.pallas_call(
        _final_body,
        grid=(_NBLK,),
        in_specs=[_row_spec(64), _row_spec(64), _row_spec(16), _row_spec(16),
                  _full_spec((16, 64)), _full_spec((1, 121)),
                  _full_spec((1, 121)), _full_spec((64, 121)),
                  _full_spec((64, 121))],
        out_specs=[_row_spec(121), _row_spec(121)],
        out_shape=[jax.ShapeDtypeStruct((N_PAD, 121), F32),
                   jax.ShapeDtypeStruct((N_PAD, 121), F32)],
    )(p0, p1, d0, d1, ex, b3e, b3o, W3e, W3o)


def _blockdiag(a):
    eye8 = jnp.eye(8, dtype=F32)
    B = (a.astype(F32)[:, :, None] * eye8[:, None, :]).reshape(64, 8)
    return jnp.pad(B, ((0, 0), (0, 8)))


def kernel(x, edge_index, W1, a_src1, a_dst1, b1, W2, a_src2, a_dst2, b2,
           W3, a_src3, a_dst3, b3):
    x = x.astype(F32)
    loop = jnp.arange(N, dtype=jnp.int32)
    src = jnp.concatenate([edge_index[0].astype(jnp.int32), loop])
    dst = jnp.concatenate([edge_index[1].astype(jnp.int32), loop])
    pad_e = E_PAD - src.shape[0]
    src = jnp.concatenate([src, jnp.full((pad_e,), N, jnp.int32)])
    dst = jnp.concatenate([dst, jnp.full((pad_e,), N, jnp.int32)])
    src = src.reshape(NC * NS, CPT, CHUNK)
    dst = dst.reshape(NC * NS, CPT, CHUNK)

    xp = jnp.pad(x, ((0, N_PAD - N), (0, 0)))
    Bs1, Bd1 = _blockdiag(a_src1), _blockdiag(a_dst1)
    Bs2, Bd2 = _blockdiag(a_src2), _blockdiag(a_dst2)
    A_s = jnp.pad(jnp.tile(a_src3.astype(F32).reshape(242, 1), (1, 8)),
                  ((0, 0), (0, 8)))
    A_d = jnp.pad(jnp.tile(a_dst3.astype(F32).reshape(242, 1), (1, 8)),
                  ((0, 0), (0, 8)))
    pv = jnp.concatenate([jnp.zeros((8,), F32),
                          jnp.full((8,), NEGH, F32)]).reshape(1, 16)
    ex8 = jnp.pad(jnp.kron(jnp.eye(8, dtype=F32), jnp.ones((1, 8), F32)),
                  ((0, 8), (0, 0)))
    ex1 = jnp.zeros((16, 64), F32).at[0, :].set(1.0)
    z16 = jnp.zeros((N_PAD, 16), F32)
    z64 = jnp.zeros((N_PAD, 64), F32)
    W3e = W3.astype(F32)[:, 0::2]
    W3o = W3.astype(F32)[:, 1::2]
    b3e = b3.astype(F32)[0::2].reshape(1, 121)
    b3o = b3.astype(F32)[1::2].reshape(1, 121)
    b1r = b1.astype(F32).reshape(1, 64)
    b2r = b2.astype(F32).reshape(1, 64)

    pass1 = _make_pass1()
    pass2_h8 = _make_pass2(True)
    pass2_h1 = _make_pass2(False)

    h1, as1, ad1 = _tables1(xp, W1.astype(F32), Bs1, Bd1, pv)
    m1 = _m16(as1, ad1)
    ee1, dp1 = pass1(src, dst, as1, ad1, m1, z16)
    op1, = pass2_h8(src, dst, ee1, h1, z64)
    h2, as2, ad2 = _tables_next(op1[0], op1[1], dp1[0], dp1[1], ex8, b1r,
                                W2.astype(F32), Bs2, Bd2, pv)
    m2 = _m16(as2, ad2)
    ee2, dp2 = pass1(src, dst, as2, ad2, m2, z16)
    op2, = pass2_h8(src, dst, ee2, h2, z64)
    x3, as3, ad3 = _tables3(op2[0], op2[1], dp2[0], dp2[1], ex8, b2r,
                            W3.astype(F32), A_s, A_d, pv)
    m3 = _m16(as3, ad3)
    ee3, dp3 = pass1(src, dst, as3, ad3, m3, z16)
    op3, = pass2_h1(src, dst, ee3, x3, z64)
    s0, s1 = _final(op3[0], op3[1], dp3[0], dp3[1], ex1, b3e, b3o, W3e, W3o)
    return jnp.stack([s0[:N], s1[:N]], axis=-1)


# speedup vs baseline: 60.0266x; 1.5013x over previous
import functools

import jax
import jax.numpy as jnp
from jax import lax
from jax.experimental import pallas as pl
from jax.experimental.pallas import tpu as pltpu
from jax.experimental.pallas import tpu_sc as plsc

N = 10000
N_PAD = 10240
NC = 2
NS = 16
CHUNK = 128
CPT = 81
E_PAD = NC * NS * CPT * CHUNK
RPS = N_PAD // NS
NEGH = -5e29
F32 = jnp.float32


def _sc_mesh():
    return plsc.VectorSubcoreMesh(
        core_axis_name="c", subcore_axis_name="s", num_cores=NC, num_subcores=NS)


def _pass1_body(src_hbm, dst_hbm, asrc_hbm, adst_hbm, m16_hbm, z16_hbm,
                ee_hbm, dpart_hbm,
                src_v, dst_v, rs0, rd0, rs1, rd1, ee_v, m16_v, den_sp,
                sem_a, sem_b):
    c = lax.axis_index("c")
    s = lax.axis_index("s")
    wid = c * NS + s
    pltpu.sync_copy(z16_hbm.at[pl.ds(s * RPS, RPS)],
                    den_sp.at[pl.ds(s * RPS, RPS)])
    pltpu.sync_copy(m16_hbm, m16_v)
    pltpu.sync_copy(src_hbm.at[wid], src_v)
    pltpu.sync_copy(dst_hbm.at[wid], dst_v)
    plsc.subcore_barrier()
    m16 = m16_v[...]

    def issue(j, rs, rd, sem):
        pltpu.async_copy(asrc_hbm.at[src_v.at[j]], rs, sem)
        pltpu.async_copy(adst_hbm.at[dst_v.at[j]], rd, sem)

    def wait(rs, rd, sem):
        pltpu.make_async_copy(asrc_hbm.at[src_v.at[0]], rs, sem).wait()
        pltpu.make_async_copy(adst_hbm.at[dst_v.at[0]], rd, sem).wait()

    def compute(j, rs, rd):
        def edge_body(k, carry2):
            e = rs[k] + rd[k]
            e = jnp.where(e > 0, e, 0.2 * e)
            ee_v[k] = jnp.exp(e - m16)
            return carry2

        lax.fori_loop(0, CHUNK, edge_body, 0, unroll=4)
        pltpu.sync_copy(ee_v, den_sp.at[dst_v.at[j]], add=True)
        pltpu.sync_copy(ee_v, ee_hbm.at[wid, j])

    issue(0, rs0, rd0, sem_a)

    def pair_body(t, carry):
        j0 = 2 * t
        issue(j0 + 1, rs1, rd1, sem_b)
        wait(rs0, rd0, sem_a)
        compute(j0, rs0, rd0)

        @pl.when(j0 + 2 < CPT)
        def _():
            issue(j0 + 2, rs0, rd0, sem_a)

        wait(rs1, rd1, sem_b)
        compute(j0 + 1, rs1, rd1)
        return carry

    lax.fori_loop(0, CPT // 2, pair_body, 0)
    wait(rs0, rd0, sem_a)
    compute(CPT - 1, rs0, rd0)
    plsc.subcore_barrier()
    pltpu.sync_copy(den_sp.at[pl.ds(s * RPS, RPS)],
                    dpart_hbm.at[c, pl.ds(s * RPS, RPS)])


def _make_pass1():
    return functools.partial(
        pl.kernel,
        out_type=[
            jax.ShapeDtypeStruct((NC * NS, CPT, CHUNK, 16), F32),
            jax.ShapeDtypeStruct((NC, N_PAD, 16), F32),
        ],
        mesh=_sc_mesh(),
        compiler_params=pltpu.CompilerParams(use_tc_tiling_on_sc=False, needs_layout_passes=False),
        scratch_types=[
            pltpu.VMEM((CPT, CHUNK), jnp.int32),
            pltpu.VMEM((CPT, CHUNK), jnp.int32),
            pltpu.VMEM((CHUNK, 16), F32),
            pltpu.VMEM((CHUNK, 16), F32),
            pltpu.VMEM((CHUNK, 16), F32),
            pltpu.VMEM((CHUNK, 16), F32),
            pltpu.VMEM((CHUNK, 16), F32),
            pltpu.VMEM((16,), F32),
            pltpu.VMEM_SHARED((N_PAD, 16), F32),
            pltpu.SemaphoreType.DMA,
            pltpu.SemaphoreType.DMA,
        ],
    )(_pass1_body)


def _make_pass2(heads8):
    def body(src_hbm, dst_hbm, ee_hbm, h_hbm, z64_hbm,
             opart_hbm,
             src_v, dst_v, ee0, hs0, ee1, hs1, msg_v, out_sp, sem_a, sem_b):
        c = lax.axis_index("c")
        s = lax.axis_index("s")
        wid = c * NS + s
        pltpu.sync_copy(z64_hbm.at[pl.ds(s * RPS, RPS)],
                        out_sp.at[pl.ds(s * RPS, RPS)])
        pltpu.sync_copy(src_hbm.at[wid], src_v)
        pltpu.sync_copy(dst_hbm.at[wid], dst_v)
        plsc.subcore_barrier()
        iota16 = lax.broadcasted_iota(jnp.int32, (16,), 0)

        def issue(j, hs, ee, sem):
            pltpu.async_copy(h_hbm.at[src_v.at[j]], hs, sem)
            pltpu.async_copy(ee_hbm.at[wid, j], ee, sem)

        def wait(hs, ee, sem):
            pltpu.make_async_copy(h_hbm.at[src_v.at[0]], hs, sem).wait()
            pltpu.make_async_copy(ee_hbm.at[wid, 0], ee, sem).wait()

        def compute(j, hs, ee):
            def edge_body(k, carry2):
                ksplat = jnp.full((16,), 0, jnp.int32) + k
                for v in range(4):
                    if heads8:
                        pat = 2 * v + jnp.where(iota16 >= 8, 1, 0)
                    else:
                        pat = iota16 * 0
                    av = plsc.load_gather(ee, [ksplat, pat])
                    msg_v[k, pl.ds(16 * v, 16)] = hs[k, pl.ds(16 * v, 16)] * av
                return carry2

            lax.fori_loop(0, CHUNK, edge_body, 0, unroll=2)
            pltpu.sync_copy(msg_v, out_sp.at[dst_v.at[j]], add=True)

        issue(0, hs0, ee0, sem_a)

        def pair_body(t, carry):
            j0 = 2 * t
            issue(j0 + 1, hs1, ee1, sem_b)
            wait(hs0, ee0, sem_a)
            compute(j0, hs0, ee0)

            @pl.when(j0 + 2 < CPT)
            def _():
                issue(j0 + 2, hs0, ee0, sem_a)

            wait(hs1, ee1, sem_b)
            compute(j0 + 1, hs1, ee1)
            return carry

        lax.fori_loop(0, CPT // 2, pair_body, 0)
        wait(hs0, ee0, sem_a)
        compute(CPT - 1, hs0, ee0)
        plsc.subcore_barrier()
        pltpu.sync_copy(out_sp.at[pl.ds(s * RPS, RPS)],
                        opart_hbm.at[c, pl.ds(s * RPS, RPS)])

    return functools.partial(
        pl.kernel,
        out_type=[jax.ShapeDtypeStruct((NC, N_PAD, 64), F32)],
        mesh=_sc_mesh(),
        compiler_params=pltpu.CompilerParams(use_tc_tiling_on_sc=False, needs_layout_passes=False),
        scratch_types=[
            pltpu.VMEM((CPT, CHUNK), jnp.int32),
            pltpu.VMEM((CPT, CHUNK), jnp.int32),
            pltpu.VMEM((CHUNK, 16), F32),
            pltpu.VMEM((CHUNK, 64), F32),
            pltpu.VMEM((CHUNK, 16), F32),
            pltpu.VMEM((CHUNK, 64), F32),
            pltpu.VMEM((CHUNK, 64), F32),
            pltpu.VMEM_SHARED((N_PAD, 64), F32),
            pltpu.SemaphoreType.DMA,
            pltpu.SemaphoreType.DMA,
        ],
    )(body)


_BLK = 512
_NBLK = N_PAD // _BLK


def _tables1_body(x_ref, w_ref, bs_ref, bd_ref, pv_ref, h_ref, as_ref, ad_ref):
    h = jnp.dot(x_ref[...], w_ref[...], preferred_element_type=F32)
    h_ref[...] = h
    as_ref[...] = jnp.dot(h, bs_ref[...], preferred_element_type=F32) + pv_ref[...]
    ad_ref[...] = jnp.dot(h, bd_ref[...], preferred_element_type=F32) + pv_ref[...]


def _tables_next_body(p0_ref, p1_ref, d0_ref, d1_ref, ex_ref, b_ref, w_ref,
                      bs_ref, bd_ref, pv_ref, h_ref, as_ref, ad_ref):
    dexp = jnp.dot(d0_ref[...] + d1_ref[...], ex_ref[...],
                   preferred_element_type=F32) + 1e-16
    xx = jax.nn.relu((p0_ref[...] + p1_ref[...]) / dexp + b_ref[...])
    h = jnp.dot(xx, w_ref[...], preferred_element_type=F32)
    h_ref[...] = h
    as_ref[...] = jnp.dot(h, bs_ref[...], preferred_element_type=F32) + pv_ref[...]
    ad_ref[...] = jnp.dot(h, bd_ref[...], preferred_element_type=F32) + pv_ref[...]


def _tables3_body(p0_ref, p1_ref, d0_ref, d1_ref, ex_ref, b_ref, w_ref,
                  as3_ref, ad3_ref, pv_ref, x_ref, as_ref, ad_ref):
    dexp = jnp.dot(d0_ref[...] + d1_ref[...], ex_ref[...],
                   preferred_element_type=F32) + 1e-16
    xx = jax.nn.relu((p0_ref[...] + p1_ref[...]) / dexp + b_ref[...])
    x_ref[...] = xx
    hw = jnp.dot(xx, w_ref[...], preferred_element_type=F32)
    as_ref[...] = jnp.dot(hw, as3_ref[...], preferred_element_type=F32) + pv_ref[...]
    ad_ref[...] = jnp.dot(hw, ad3_ref[...], preferred_element_type=F32) + pv_ref[...]


def _m16_body(as_ref, ad_ref, o_ref):
    m = jnp.max(as_ref[...], axis=0) + jnp.max(ad_ref[...], axis=0)
    m = jnp.where(m > 0, m, 0.2 * m)
    o_ref[...] = jnp.broadcast_to(m[None, :], (8, 16))


def _final_body(p0_ref, p1_ref, d0_ref, d1_ref, ex_ref, b3e_ref, b3o_ref,
                w3e_ref, w3o_ref, s0_ref, s1_ref):
    dexp = jnp.dot(d0_ref[...] + d1_ref[...], ex_ref[...],
                   preferred_element_type=F32) + 1e-16
    agg = (p0_ref[...] + p1_ref[...]) / dexp
    z0 = jax.nn.relu(jnp.dot(agg, w3e_ref[...], preferred_element_type=F32)
                     + b3e_ref[...])
    z1 = jax.nn.relu(jnp.dot(agg, w3o_ref[...], preferred_element_type=F32)
                     + b3o_ref[...])
    m = jnp.maximum(z0, z1)
    e0 = jnp.exp(z0 - m)
    e1 = jnp.exp(z1 - m)
    t = e0 + e1
    s0_ref[...] = e0 / t
    s1_ref[...] = e1 / t


def _row_spec(cols):
    return pl.BlockSpec((_BLK, cols), lambda i: (i, 0))


def _full_spec(shape):
    return pl.BlockSpec(shape, lambda i: tuple(0 for _ in shape))


def _tables1(xp, W1, Bs, Bd, pv):
    return pl.pallas_call(
        _tables1_body,
        grid=(_NBLK,),
        in_specs=[_row_spec(128), _full_spec((128, 64)), _full_spec((64, 16)),
                  _full_spec((64, 16)), _full_spec((1, 16))],
        out_specs=[_row_spec(64), _row_spec(16), _row_spec(16)],
        out_shape=[jax.ShapeDtypeStruct((N_PAD, 64), F32),
                   jax.ShapeDtypeStruct((N_PAD, 16), F32),
                   jax.ShapeDtypeStruct((N_PAD, 16), F32)],
    )(xp, W1, Bs, Bd, pv)


def _tables_next(p0, p1, d0, d1, ex, b, W, Bs, Bd, pv):
    return pl.pallas_call(
        _tables_next_body,
        grid=(_NBLK,),
        in_specs=[_row_spec(64), _row_spec(64), _row_spec(16), _row_spec(16),
                  _full_spec((16, 64)), _full_spec((1, 64)),
                  _full_spec((64, 64)), _full_spec((64, 16)),
                  _full_spec((64, 16)), _full_spec((1, 16))],
        out_specs=[_row_spec(64), _row_spec(16), _row_spec(16)],
        out_shape=[jax.ShapeDtypeStruct((N_PAD, 64), F32),
                   jax.ShapeDtypeStruct((N_PAD, 16), F32),
                   jax.ShapeDtypeStruct((N_PAD, 16), F32)],
    )(p0, p1, d0, d1, ex, b, W, Bs, Bd, pv)


def _tables3(p0, p1, d0, d1, ex, b, W3, A_s, A_d, pv):
    return pl.pallas_call(
        _tables3_body,
        grid=(_NBLK,),
        in_specs=[_row_spec(64), _row_spec(64), _row_spec(16), _row_spec(16),
                  _full_spec((16, 64)), _full_spec((1, 64)),
                  _full_spec((64, 242)), _full_spec((242, 16)),
                  _full_spec((242, 16)), _full_spec((1, 16))],
        out_specs=[_row_spec(64), _row_spec(16), _row_spec(16)],
        out_shape=[jax.ShapeDtypeStruct((N_PAD, 64), F32),
                   jax.ShapeDtypeStruct((N_PAD, 16), F32),
                   jax.ShapeDtypeStruct((N_PAD, 16), F32)],
    )(p0, p1, d0, d1, ex, b, W3, A_s, A_d, pv)


def _m16(asrc, adst):
    out = pl.pallas_call(
        _m16_body,
        out_shape=jax.ShapeDtypeStruct((8, 16), F32),
    )(asrc, adst)
    return out[0]


def _final(p0, p1, d0, d1, ex, b3e, b3o, W3e, W3o):
    return pl.pallas_call(
        _final_body,
        grid=(_NBLK,),
        in_specs=[_row_spec(64), _row_spec(64), _row_spec(16), _row_spec(16),
                  _full_spec((16, 64)), _full_spec((1, 121)),
                  _full_spec((1, 121)), _full_spec((64, 121)),
                  _full_spec((64, 121))],
        out_specs=[_row_spec(121), _row_spec(121)],
        out_shape=[jax.ShapeDtypeStruct((N_PAD, 121), F32),
                   jax.ShapeDtypeStruct((N_PAD, 121), F32)],
    )(p0, p1, d0, d1, ex, b3e, b3o, W3e, W3o)


def _blockdiag(a):
    eye8 = jnp.eye(8, dtype=F32)
    B = (a.astype(F32)[:, :, None] * eye8[:, None, :]).reshape(64, 8)
    return jnp.pad(B, ((0, 0), (0, 8)))


def kernel(x, edge_index, W1, a_src1, a_dst1, b1, W2, a_src2, a_dst2, b2,
           W3, a_src3, a_dst3, b3):
    x = x.astype(F32)
    loop = jnp.arange(N, dtype=jnp.int32)
    src = jnp.concatenate([edge_index[0].astype(jnp.int32), loop])
    dst = jnp.concatenate([edge_index[1].astype(jnp.int32), loop])
    pad_e = E_PAD - src.shape[0]
    src = jnp.concatenate([src, jnp.full((pad_e,), N, jnp.int32)])
    dst = jnp.concatenate([dst, jnp.full((pad_e,), N, jnp.int32)])
    src = src.reshape(NC * NS, CPT, CHUNK)
    dst = dst.reshape(NC * NS, CPT, CHUNK)

    xp = jnp.pad(x, ((0, N_PAD - N), (0, 0)))
    Bs1, Bd1 = _blockdiag(a_src1), _blockdiag(a_dst1)
    Bs2, Bd2 = _blockdiag(a_src2), _blockdiag(a_dst2)
    A_s = jnp.pad(jnp.tile(a_src3.astype(F32).reshape(242, 1), (1, 8)),
                  ((0, 0), (0, 8)))
    A_d = jnp.pad(jnp.tile(a_dst3.astype(F32).reshape(242, 1), (1, 8)),
                  ((0, 0), (0, 8)))
    pv = jnp.concatenate([jnp.zeros((8,), F32),
                          jnp.full((8,), NEGH, F32)]).reshape(1, 16)
    ex8 = jnp.pad(jnp.kron(jnp.eye(8, dtype=F32), jnp.ones((1, 8), F32)),
                  ((0, 8), (0, 0)))
    ex1 = jnp.zeros((16, 64), F32).at[0, :].set(1.0)
    z16 = jnp.zeros((N_PAD, 16), F32)
    z64 = jnp.zeros((N_PAD, 64), F32)
    W3e = W3.astype(F32)[:, 0::2]
    W3o = W3.astype(F32)[:, 1::2]
    b3e = b3.astype(F32)[0::2].reshape(1, 121)
    b3o = b3.astype(F32)[1::2].reshape(1, 121)
    b1r = b1.astype(F32).reshape(1, 64)
    b2r = b2.astype(F32).reshape(1, 64)

    pass1 = _make_pass1()
    pass2_h8 = _make_pass2(True)
    pass2_h1 = _make_pass2(False)

    h1, as1, ad1 = _tables1(xp, W1.astype(F32), Bs1, Bd1, pv)
    m1 = _m16(as1, ad1)
    ee1, dp1 = pass1(src, dst, as1, ad1, m1, z16)
    op1, = pass2_h8(src, dst, ee1, h1, z64)
    h2, as2, ad2 = _tables_next(op1[0], op1[1], dp1[0], dp1[1], ex8, b1r,
                                W2.astype(F32), Bs2, Bd2, pv)
    m2 = _m16(as2, ad2)
    ee2, dp2 = pass1(src, dst, as2, ad2, m2, z16)
    op2, = pass2_h8(src, dst, ee2, h2, z64)
    x3, as3, ad3 = _tables3(op2[0], op2[1], dp2[0], dp2[1], ex8, b2r,
                            W3.astype(F32), A_s, A_d, pv)
    m3 = _m16(as3, ad3)
    ee3, dp3 = pass1(src, dst, as3, ad3, m3, z16)
    op3, = pass2_h1(src, dst, ee3, x3, z64)
    s0, s1 = _final(op3[0], op3[1], dp3[0], dp3[1], ex1, b3e, b3o, W3e, W3o)
    return jnp.stack([s0[:N], s1[:N]], axis=-1)


# speedup vs baseline: 62.4970x; 1.0412x over previous
import functools

import jax
import jax.numpy as jnp
from jax import lax
from jax.experimental import pallas as pl
from jax.experimental.pallas import tpu as pltpu
from jax.experimental.pallas import tpu_sc as plsc

N = 10000
N_PAD = 10240
NC = 2
NS = 16
CHUNK = 128
CPT = 81
E_PAD = NC * NS * CPT * CHUNK
RPS = N_PAD // NS
NEGH = -5e29
F32 = jnp.float32


def _sc_mesh():
    return plsc.VectorSubcoreMesh(
        core_axis_name="c", subcore_axis_name="s", num_cores=NC, num_subcores=NS)


def _make_edge(heads8):
    def body(src_hbm, dst_hbm, big_hbm, adst_hbm, m16_hbm, z80_hbm,
             opart_hbm,
             src_v, dst_v, gs0, gd0, gs1, gd1, cm_v, m16_v, acc_sp,
             sem_a, sem_b):
        c = lax.axis_index("c")
        s = lax.axis_index("s")
        wid = c * NS + s
        pltpu.sync_copy(z80_hbm.at[pl.ds(s * RPS, RPS)],
                        acc_sp.at[pl.ds(s * RPS, RPS)])
        pltpu.sync_copy(m16_hbm, m16_v)
        pltpu.sync_copy(src_hbm.at[wid], src_v)
        pltpu.sync_copy(dst_hbm.at[wid], dst_v)
        plsc.subcore_barrier()
        m16 = m16_v[...]
        iota16 = lax.broadcasted_iota(jnp.int32, (16,), 0)

        def issue(j, gs, gd, sem):
            pltpu.async_copy(big_hbm.at[src_v.at[j]], gs, sem)
            pltpu.async_copy(adst_hbm.at[dst_v.at[j]], gd, sem)

        def wait(gs, gd, sem):
            pltpu.make_async_copy(big_hbm.at[src_v.at[0]], gs, sem).wait()
            pltpu.make_async_copy(adst_hbm.at[dst_v.at[0]], gd, sem).wait()

        def compute(j, gs, gd):
            def edge_body(k, carry2):
                e = gs[k, pl.ds(0, 16)] + gd[k]
                e = jnp.where(e > 0, e, 0.2 * e)
                cm_v[k, pl.ds(64, 16)] = jnp.exp(e - m16)
                ksplat = jnp.full((16,), 0, jnp.int32) + k
                for v in range(4):
                    if heads8:
                        pat = 64 + 2 * v + jnp.where(iota16 >= 8, 1, 0)
                    else:
                        pat = 64 + iota16 * 0
                    av = plsc.load_gather(cm_v, [ksplat, pat])
                    cm_v[k, pl.ds(16 * v, 16)] = (
                        gs[k, pl.ds(16 + 16 * v, 16)] * av)
                return carry2

            lax.fori_loop(0, CHUNK, edge_body, 0, unroll=2)
            pltpu.sync_copy(cm_v, acc_sp.at[dst_v.at[j]], add=True)

        issue(0, gs0, gd0, sem_a)

        def pair_body(t, carry):
            j0 = 2 * t
            issue(j0 + 1, gs1, gd1, sem_b)
            wait(gs0, gd0, sem_a)
            compute(j0, gs0, gd0)

            @pl.when(j0 + 2 < CPT)
            def _():
                issue(j0 + 2, gs0, gd0, sem_a)

            wait(gs1, gd1, sem_b)
            compute(j0 + 1, gs1, gd1)
            return carry

        lax.fori_loop(0, CPT // 2, pair_body, 0)
        wait(gs0, gd0, sem_a)
        compute(CPT - 1, gs0, gd0)
        plsc.subcore_barrier()
        pltpu.sync_copy(acc_sp.at[pl.ds(s * RPS, RPS)],
                        opart_hbm.at[c, pl.ds(s * RPS, RPS)])

    return functools.partial(
        pl.kernel,
        out_type=[jax.ShapeDtypeStruct((NC, N_PAD, 80), F32)],
        mesh=_sc_mesh(),
        compiler_params=pltpu.CompilerParams(use_tc_tiling_on_sc=False,
                                             needs_layout_passes=False),
        scratch_types=[
            pltpu.VMEM((CPT, CHUNK), jnp.int32),
            pltpu.VMEM((CPT, CHUNK), jnp.int32),
            pltpu.VMEM((CHUNK, 80), F32),
            pltpu.VMEM((CHUNK, 16), F32),
            pltpu.VMEM((CHUNK, 80), F32),
            pltpu.VMEM((CHUNK, 16), F32),
            pltpu.VMEM((CHUNK, 80), F32),
            pltpu.VMEM((16,), F32),
            pltpu.VMEM_SHARED((N_PAD, 80), F32),
            pltpu.SemaphoreType.DMA,
            pltpu.SemaphoreType.DMA,
        ],
    )(body)


_BLK = 512
_NBLK = N_PAD // _BLK


def _tables1_body(x_ref, w_ref, bs_ref, bd_ref, pv_ref,
                  big_ref, as_ref, ad_ref):
    h = jnp.dot(x_ref[...], w_ref[...], preferred_element_type=F32)
    asrc = jnp.dot(h, bs_ref[...], preferred_element_type=F32) + pv_ref[...]
    big_ref[...] = jnp.concatenate([asrc, h], axis=1)
    as_ref[...] = asrc
    ad_ref[...] = jnp.dot(h, bd_ref[...], preferred_element_type=F32) + pv_ref[...]


def _tables_next_body(p0_ref, p1_ref, ex_ref, b_ref, w_ref, bs_ref, bd_ref,
                      pv_ref, big_ref, as_ref, ad_ref):
    p = p0_ref[...] + p1_ref[...]
    dexp = jnp.dot(p[:, 64:80], ex_ref[...], preferred_element_type=F32) + 1e-16
    xx = jax.nn.relu(p[:, 0:64] / dexp + b_ref[...])
    h = jnp.dot(xx, w_ref[...], preferred_element_type=F32)
    asrc = jnp.dot(h, bs_ref[...], preferred_element_type=F32) + pv_ref[...]
    big_ref[...] = jnp.concatenate([asrc, h], axis=1)
    as_ref[...] = asrc
    ad_ref[...] = jnp.dot(h, bd_ref[...], preferred_element_type=F32) + pv_ref[...]


def _tables3_body(p0_ref, p1_ref, ex_ref, b_ref, w_ref, as3_ref, ad3_ref,
                  pv_ref, big_ref, as_ref, ad_ref):
    p = p0_ref[...] + p1_ref[...]
    dexp = jnp.dot(p[:, 64:80], ex_ref[...], preferred_element_type=F32) + 1e-16
    xx = jax.nn.relu(p[:, 0:64] / dexp + b_ref[...])
    hw = jnp.dot(xx, w_ref[...], preferred_element_type=F32)
    asrc = jnp.dot(hw, as3_ref[...], preferred_element_type=F32) + pv_ref[...]
    big_ref[...] = jnp.concatenate([asrc, xx], axis=1)
    as_ref[...] = asrc
    ad_ref[...] = jnp.dot(hw, ad3_ref[...], preferred_element_type=F32) + pv_ref[...]


def _m16_body(as_ref, ad_ref, o_ref):
    m = jnp.max(as_ref[...], axis=0) + jnp.max(ad_ref[...], axis=0)
    m = jnp.where(m > 0, m, 0.2 * m)
    o_ref[...] = jnp.broadcast_to(m[None, :], (8, 16))


def _final_body(p0_ref, p1_ref, ex_ref, b3e_ref, b3o_ref, w3e_ref, w3o_ref,
                s0_ref, s1_ref):
    p = p0_ref[...] + p1_ref[...]
    dexp = jnp.dot(p[:, 64:80], ex_ref[...], preferred_element_type=F32) + 1e-16
    agg = p[:, 0:64] / dexp
    z0 = jax.nn.relu(jnp.dot(agg, w3e_ref[...], preferred_element_type=F32)
                     + b3e_ref[...])
    z1 = jax.nn.relu(jnp.dot(agg, w3o_ref[...], preferred_element_type=F32)
                     + b3o_ref[...])
    m = jnp.maximum(z0, z1)
    e0 = jnp.exp(z0 - m)
    e1 = jnp.exp(z1 - m)
    t = e0 + e1
    s0_ref[...] = e0 / t
    s1_ref[...] = e1 / t


def _row_spec(cols):
    return pl.BlockSpec((_BLK, cols), lambda i: (i, 0))


def _full_spec(shape):
    return pl.BlockSpec(shape, lambda i: tuple(0 for _ in shape))


_TBL_OUT = [jax.ShapeDtypeStruct((N_PAD, 80), F32),
            jax.ShapeDtypeStruct((N_PAD, 16), F32),
            jax.ShapeDtypeStruct((N_PAD, 16), F32)]
_TBL_OUT_SPECS = [_row_spec(80), _row_spec(16), _row_spec(16)]


def _tables1(xp, W1, Bs, Bd, pv):
    return pl.pallas_call(
        _tables1_body,
        grid=(_NBLK,),
        in_specs=[_row_spec(128), _full_spec((128, 64)), _full_spec((64, 16)),
                  _full_spec((64, 16)), _full_spec((1, 16))],
        out_specs=_TBL_OUT_SPECS,
        out_shape=_TBL_OUT,
    )(xp, W1, Bs, Bd, pv)


def _tables_next(p0, p1, ex, b, W, Bs, Bd, pv):
    return pl.pallas_call(
        _tables_next_body,
        grid=(_NBLK,),
        in_specs=[_row_spec(80), _row_spec(80), _full_spec((16, 64)),
                  _full_spec((1, 64)), _full_spec((64, 64)),
                  _full_spec((64, 16)), _full_spec((64, 16)),
                  _full_spec((1, 16))],
        out_specs=_TBL_OUT_SPECS,
        out_shape=_TBL_OUT,
    )(p0, p1, ex, b, W, Bs, Bd, pv)


def _tables3(p0, p1, ex, b, W3, A_s, A_d, pv):
    return pl.pallas_call(
        _tables3_body,
        grid=(_NBLK,),
        in_specs=[_row_spec(80), _row_spec(80), _full_spec((16, 64)),
                  _full_spec((1, 64)), _full_spec((64, 242)),
                  _full_spec((242, 16)), _full_spec((242, 16)),
                  _full_spec((1, 16))],
        out_specs=_TBL_OUT_SPECS,
        out_shape=_TBL_OUT,
    )(p0, p1, ex, b, W3, A_s, A_d, pv)


def _m16(asrc, adst):
    out = pl.pallas_call(
        _m16_body,
        out_shape=jax.ShapeDtypeStruct((8, 16), F32),
    )(asrc, adst)
    return out[0]


def _final(p0, p1, ex, b3e, b3o, W3e, W3o):
    return pl.pallas_call(
        _final_body,
        grid=(_NBLK,),
        in_specs=[_row_spec(80), _row_spec(80), _full_spec((16, 64)),
                  _full_spec((1, 121)), _full_spec((1, 121)),
                  _full_spec((64, 121)), _full_spec((64, 121))],
        out_specs=[_row_spec(121), _row_spec(121)],
        out_shape=[jax.ShapeDtypeStruct((N_PAD, 121), F32),
                   jax.ShapeDtypeStruct((N_PAD, 121), F32)],
    )(p0, p1, ex, b3e, b3o, W3e, W3o)


def _blockdiag(a):
    eye8 = jnp.eye(8, dtype=F32)
    B = (a.astype(F32)[:, :, None] * eye8[:, None, :]).reshape(64, 8)
    return jnp.pad(B, ((0, 0), (0, 8)))


def kernel(x, edge_index, W1, a_src1, a_dst1, b1, W2, a_src2, a_dst2, b2,
           W3, a_src3, a_dst3, b3):
    x = x.astype(F32)
    loop = jnp.arange(N, dtype=jnp.int32)
    src = jnp.concatenate([edge_index[0].astype(jnp.int32), loop])
    dst = jnp.concatenate([edge_index[1].astype(jnp.int32), loop])
    pad_e = E_PAD - src.shape[0]
    src = jnp.concatenate([src, jnp.full((pad_e,), N, jnp.int32)])
    dst = jnp.concatenate([dst, jnp.full((pad_e,), N, jnp.int32)])
    src = src.reshape(NC * NS, CPT, CHUNK)
    dst = dst.reshape(NC * NS, CPT, CHUNK)

    xp = jnp.pad(x, ((0, N_PAD - N), (0, 0)))
    Bs1, Bd1 = _blockdiag(a_src1), _blockdiag(a_dst1)
    Bs2, Bd2 = _blockdiag(a_src2), _blockdiag(a_dst2)
    A_s = jnp.pad(jnp.tile(a_src3.astype(F32).reshape(242, 1), (1, 8)),
                  ((0, 0), (0, 8)))
    A_d = jnp.pad(jnp.tile(a_dst3.astype(F32).reshape(242, 1), (1, 8)),
                  ((0, 0), (0, 8)))
    pv = jnp.concatenate([jnp.zeros((8,), F32),
                          jnp.full((8,), NEGH, F32)]).reshape(1, 16)
    ex8 = jnp.pad(jnp.kron(jnp.eye(8, dtype=F32), jnp.ones((1, 8), F32)),
                  ((0, 8), (0, 0)))
    ex1 = jnp.zeros((16, 64), F32).at[0, :].set(1.0)
    z80 = jnp.zeros((N_PAD, 80), F32)
    W3e = W3.astype(F32)[:, 0::2]
    W3o = W3.astype(F32)[:, 1::2]
    b3e = b3.astype(F32)[0::2].reshape(1, 121)
    b3o = b3.astype(F32)[1::2].reshape(1, 121)
    b1r = b1.astype(F32).reshape(1, 64)
    b2r = b2.astype(F32).reshape(1, 64)

    edge_h8 = _make_edge(True)
    edge_h1 = _make_edge(False)

    big1, as1, ad1 = _tables1(xp, W1.astype(F32), Bs1, Bd1, pv)
    m1 = _m16(as1, ad1)
    op1, = edge_h8(src, dst, big1, ad1, m1, z80)
    big2, as2, ad2 = _tables_next(op1[0], op1[1], ex8, b1r, W2.astype(F32),
                                  Bs2, Bd2, pv)
    m2 = _m16(as2, ad2)
    op2, = edge_h8(src, dst, big2, ad2, m2, z80)
    big3, as3, ad3 = _tables3(op2[0], op2[1], ex8, b2r, W3.astype(F32),
                              A_s, A_d, pv)
    m3 = _m16(as3, ad3)
    op3, = edge_h1(src, dst, big3, ad3, m3, z80)
    s0, s1 = _final(op3[0], op3[1], ex1, b3e, b3o, W3e, W3o)
    return jnp.stack([s0[:N], s1[:N]], axis=-1)


# speedup vs baseline: 66.0345x; 1.0566x over previous
import functools

import jax
import jax.numpy as jnp
from jax import lax
from jax.experimental import pallas as pl
from jax.experimental.pallas import tpu as pltpu
from jax.experimental.pallas import tpu_sc as plsc

N = 10000
N_PAD = 10240
NC = 2
NS = 16
CHUNK = 128
CPT = 81
E_PAD = NC * NS * CPT * CHUNK
RPS = N_PAD // NS
NEGH = -5e29
F32 = jnp.float32


def _sc_mesh():
    return plsc.VectorSubcoreMesh(
        core_axis_name="c", subcore_axis_name="s", num_cores=NC, num_subcores=NS)


def _make_edge(heads8):
    def body(src_hbm, dst_hbm, big_hbm, adst_hbm, m16_hbm, z80_hbm,
             opart_hbm,
             src_v, dst_v, gs0, gd0, gs1, gd1, cm0, cm1, m16_v, acc_sp,
             sem_a, sem_b, sem_ca, sem_cb):
        c = lax.axis_index("c")
        s = lax.axis_index("s")
        wid = c * NS + s
        pltpu.sync_copy(z80_hbm.at[pl.ds(s * RPS, RPS)],
                        acc_sp.at[pl.ds(s * RPS, RPS)])
        pltpu.sync_copy(m16_hbm, m16_v)
        pltpu.sync_copy(src_hbm.at[wid], src_v)
        pltpu.sync_copy(dst_hbm.at[wid], dst_v)
        plsc.subcore_barrier()
        m16 = m16_v[...]
        iota16 = lax.broadcasted_iota(jnp.int32, (16,), 0)

        def issue(j, gs, gd, sem):
            pltpu.async_copy(big_hbm.at[src_v.at[j]], gs, sem)
            pltpu.async_copy(adst_hbm.at[dst_v.at[j]], gd, sem)

        def wait(gs, gd, sem):
            pltpu.make_async_copy(big_hbm.at[src_v.at[0]], gs, sem).wait()
            pltpu.make_async_copy(adst_hbm.at[dst_v.at[0]], gd, sem).wait()

        def compute(j, gs, gd, cm, sem_c):
            def edge_body(k, carry2):
                e = gs[k, pl.ds(0, 16)] + gd[k]
                e = jnp.where(e > 0, e, 0.2 * e)
                cm[k, pl.ds(64, 16)] = jnp.exp(e - m16)
                ksplat = jnp.full((16,), 0, jnp.int32) + k
                for v in range(4):
                    if heads8:
                        pat = 64 + 2 * v + jnp.where(iota16 >= 8, 1, 0)
                    else:
                        pat = 64 + iota16 * 0
                    av = plsc.load_gather(cm, [ksplat, pat])
                    cm[k, pl.ds(16 * v, 16)] = (
                        gs[k, pl.ds(16 + 16 * v, 16)] * av)
                return carry2

            lax.fori_loop(0, CHUNK, edge_body, 0, unroll=2)
            pltpu.async_copy(cm, acc_sp.at[dst_v.at[j]], sem_c, add=True)

        def drain(cm, sem_c):
            pltpu.make_async_copy(cm, acc_sp.at[dst_v.at[0]], sem_c).wait()

        issue(0, gs0, gd0, sem_a)

        def pair_body(t, carry):
            j0 = 2 * t
            issue(j0 + 1, gs1, gd1, sem_b)
            wait(gs0, gd0, sem_a)

            @pl.when(j0 >= 2)
            def _():
                drain(cm0, sem_ca)

            compute(j0, gs0, gd0, cm0, sem_ca)

            @pl.when(j0 + 2 < CPT)
            def _():
                issue(j0 + 2, gs0, gd0, sem_a)

            wait(gs1, gd1, sem_b)

            @pl.when(j0 >= 2)
            def _():
                drain(cm1, sem_cb)

            compute(j0 + 1, gs1, gd1, cm1, sem_cb)
            return carry

        lax.fori_loop(0, CPT // 2, pair_body, 0)
        wait(gs0, gd0, sem_a)
        drain(cm0, sem_ca)
        compute(CPT - 1, gs0, gd0, cm0, sem_ca)
        drain(cm0, sem_ca)
        drain(cm1, sem_cb)
        plsc.subcore_barrier()
        pltpu.sync_copy(acc_sp.at[pl.ds(s * RPS, RPS)],
                        opart_hbm.at[c, pl.ds(s * RPS, RPS)])

    return functools.partial(
        pl.kernel,
        out_type=[jax.ShapeDtypeStruct((NC, N_PAD, 80), F32)],
        mesh=_sc_mesh(),
        compiler_params=pltpu.CompilerParams(use_tc_tiling_on_sc=False,
                                             needs_layout_passes=False),
        scratch_types=[
            pltpu.VMEM((CPT, CHUNK), jnp.int32),
            pltpu.VMEM((CPT, CHUNK), jnp.int32),
            pltpu.VMEM((CHUNK, 80), F32),
            pltpu.VMEM((CHUNK, 16), F32),
            pltpu.VMEM((CHUNK, 80), F32),
            pltpu.VMEM((CHUNK, 16), F32),
            pltpu.VMEM((CHUNK, 80), F32),
            pltpu.VMEM((CHUNK, 80), F32),
            pltpu.VMEM((16,), F32),
            pltpu.VMEM_SHARED((N_PAD, 80), F32),
            pltpu.SemaphoreType.DMA,
            pltpu.SemaphoreType.DMA,
            pltpu.SemaphoreType.DMA,
            pltpu.SemaphoreType.DMA,
        ],
    )(body)


_BLK = 512
_NBLK = N_PAD // _BLK


def _tables1_body(x_ref, w_ref, bs_ref, bd_ref, pv_ref,
                  big_ref, as_ref, ad_ref):
    h = jnp.dot(x_ref[...], w_ref[...], preferred_element_type=F32)
    asrc = jnp.dot(h, bs_ref[...], preferred_element_type=F32) + pv_ref[...]
    big_ref[...] = jnp.concatenate([asrc, h], axis=1)
    as_ref[...] = asrc
    ad_ref[...] = jnp.dot(h, bd_ref[...], preferred_element_type=F32) + pv_ref[...]


def _tables_next_body(p0_ref, p1_ref, ex_ref, b_ref, w_ref, bs_ref, bd_ref,
                      pv_ref, big_ref, as_ref, ad_ref):
    p = p0_ref[...] + p1_ref[...]
    dexp = jnp.dot(p[:, 64:80], ex_ref[...], preferred_element_type=F32) + 1e-16
    xx = jax.nn.relu(p[:, 0:64] / dexp + b_ref[...])
    h = jnp.dot(xx, w_ref[...], preferred_element_type=F32)
    asrc = jnp.dot(h, bs_ref[...], preferred_element_type=F32) + pv_ref[...]
    big_ref[...] = jnp.concatenate([asrc, h], axis=1)
    as_ref[...] = asrc
    ad_ref[...] = jnp.dot(h, bd_ref[...], preferred_element_type=F32) + pv_ref[...]


def _tables3_body(p0_ref, p1_ref, ex_ref, b_ref, w_ref, as3_ref, ad3_ref,
                  pv_ref, big_ref, as_ref, ad_ref):
    p = p0_ref[...] + p1_ref[...]
    dexp = jnp.dot(p[:, 64:80], ex_ref[...], preferred_element_type=F32) + 1e-16
    xx = jax.nn.relu(p[:, 0:64] / dexp + b_ref[...])
    hw = jnp.dot(xx, w_ref[...], preferred_element_type=F32)
    asrc = jnp.dot(hw, as3_ref[...], preferred_element_type=F32) + pv_ref[...]
    big_ref[...] = jnp.concatenate([asrc, xx], axis=1)
    as_ref[...] = asrc
    ad_ref[...] = jnp.dot(hw, ad3_ref[...], preferred_element_type=F32) + pv_ref[...]


def _m16_body(as_ref, ad_ref, o_ref):
    m = jnp.max(as_ref[...], axis=0) + jnp.max(ad_ref[...], axis=0)
    m = jnp.where(m > 0, m, 0.2 * m)
    o_ref[...] = jnp.broadcast_to(m[None, :], (8, 16))


def _final_body(p0_ref, p1_ref, ex_ref, b3e_ref, b3o_ref, w3e_ref, w3o_ref,
                s0_ref, s1_ref):
    p = p0_ref[...] + p1_ref[...]
    dexp = jnp.dot(p[:, 64:80], ex_ref[...], preferred_element_type=F32) + 1e-16
    agg = p[:, 0:64] / dexp
    z0 = jax.nn.relu(jnp.dot(agg, w3e_ref[...], preferred_element_type=F32)
                     + b3e_ref[...])
    z1 = jax.nn.relu(jnp.dot(agg, w3o_ref[...], preferred_element_type=F32)
                     + b3o_ref[...])
    m = jnp.maximum(z0, z1)
    e0 = jnp.exp(z0 - m)
    e1 = jnp.exp(z1 - m)
    t = e0 + e1
    s0_ref[...] = e0 / t
    s1_ref[...] = e1 / t


def _row_spec(cols):
    return pl.BlockSpec((_BLK, cols), lambda i: (i, 0))


def _full_spec(shape):
    return pl.BlockSpec(shape, lambda i: tuple(0 for _ in shape))


_TBL_OUT = [jax.ShapeDtypeStruct((N_PAD, 80), F32),
            jax.ShapeDtypeStruct((N_PAD, 16), F32),
            jax.ShapeDtypeStruct((N_PAD, 16), F32)]
_TBL_OUT_SPECS = [_row_spec(80), _row_spec(16), _row_spec(16)]


def _tables1(xp, W1, Bs, Bd, pv):
    return pl.pallas_call(
        _tables1_body,
        grid=(_NBLK,),
        in_specs=[_row_spec(128), _full_spec((128, 64)), _full_spec((64, 16)),
                  _full_spec((64, 16)), _full_spec((1, 16))],
        out_specs=_TBL_OUT_SPECS,
        out_shape=_TBL_OUT,
    )(xp, W1, Bs, Bd, pv)


def _tables_next(p0, p1, ex, b, W, Bs, Bd, pv):
    return pl.pallas_call(
        _tables_next_body,
        grid=(_NBLK,),
        in_specs=[_row_spec(80), _row_spec(80), _full_spec((16, 64)),
                  _full_spec((1, 64)), _full_spec((64, 64)),
                  _full_spec((64, 16)), _full_spec((64, 16)),
                  _full_spec((1, 16))],
        out_specs=_TBL_OUT_SPECS,
        out_shape=_TBL_OUT,
    )(p0, p1, ex, b, W, Bs, Bd, pv)


def _tables3(p0, p1, ex, b, W3, A_s, A_d, pv):
    return pl.pallas_call(
        _tables3_body,
        grid=(_NBLK,),
        in_specs=[_row_spec(80), _row_spec(80), _full_spec((16, 64)),
                  _full_spec((1, 64)), _full_spec((64, 242)),
                  _full_spec((242, 16)), _full_spec((242, 16)),
                  _full_spec((1, 16))],
        out_specs=_TBL_OUT_SPECS,
        out_shape=_TBL_OUT,
    )(p0, p1, ex, b, W3, A_s, A_d, pv)


def _m16(asrc, adst):
    out = pl.pallas_call(
        _m16_body,
        out_shape=jax.ShapeDtypeStruct((8, 16), F32),
    )(asrc, adst)
    return out[0]


def _final(p0, p1, ex, b3e, b3o, W3e, W3o):
    return pl.pallas_call(
        _final_body,
        grid=(_NBLK,),
        in_specs=[_row_spec(80), _row_spec(80), _full_spec((16, 64)),
                  _full_spec((1, 121)), _full_spec((1, 121)),
                  _full_spec((64, 121)), _full_spec((64, 121))],
        out_specs=[_row_spec(121), _row_spec(121)],
        out_shape=[jax.ShapeDtypeStruct((N_PAD, 121), F32),
                   jax.ShapeDtypeStruct((N_PAD, 121), F32)],
    )(p0, p1, ex, b3e, b3o, W3e, W3o)


def _blockdiag(a):
    eye8 = jnp.eye(8, dtype=F32)
    B = (a.astype(F32)[:, :, None] * eye8[:, None, :]).reshape(64, 8)
    return jnp.pad(B, ((0, 0), (0, 8)))


def kernel(x, edge_index, W1, a_src1, a_dst1, b1, W2, a_src2, a_dst2, b2,
           W3, a_src3, a_dst3, b3):
    x = x.astype(F32)
    loop = jnp.arange(N, dtype=jnp.int32)
    src = jnp.concatenate([edge_index[0].astype(jnp.int32), loop])
    dst = jnp.concatenate([edge_index[1].astype(jnp.int32), loop])
    pad_e = E_PAD - src.shape[0]
    src = jnp.concatenate([src, jnp.full((pad_e,), N, jnp.int32)])
    dst = jnp.concatenate([dst, jnp.full((pad_e,), N, jnp.int32)])
    src = src.reshape(NC * NS, CPT, CHUNK)
    dst = dst.reshape(NC * NS, CPT, CHUNK)

    xp = jnp.pad(x, ((0, N_PAD - N), (0, 0)))
    Bs1, Bd1 = _blockdiag(a_src1), _blockdiag(a_dst1)
    Bs2, Bd2 = _blockdiag(a_src2), _blockdiag(a_dst2)
    A_s = jnp.pad(jnp.tile(a_src3.astype(F32).reshape(242, 1), (1, 8)),
                  ((0, 0), (0, 8)))
    A_d = jnp.pad(jnp.tile(a_dst3.astype(F32).reshape(242, 1), (1, 8)),
                  ((0, 0), (0, 8)))
    pv = jnp.concatenate([jnp.zeros((8,), F32),
                          jnp.full((8,), NEGH, F32)]).reshape(1, 16)
    ex8 = jnp.pad(jnp.kron(jnp.eye(8, dtype=F32), jnp.ones((1, 8), F32)),
                  ((0, 8), (0, 0)))
    ex1 = jnp.zeros((16, 64), F32).at[0, :].set(1.0)
    z80 = jnp.zeros((N_PAD, 80), F32)
    W3e = W3.astype(F32)[:, 0::2]
    W3o = W3.astype(F32)[:, 1::2]
    b3e = b3.astype(F32)[0::2].reshape(1, 121)
    b3o = b3.astype(F32)[1::2].reshape(1, 121)
    b1r = b1.astype(F32).reshape(1, 64)
    b2r = b2.astype(F32).reshape(1, 64)

    edge_h8 = _make_edge(True)
    edge_h1 = _make_edge(False)

    big1, as1, ad1 = _tables1(xp, W1.astype(F32), Bs1, Bd1, pv)
    m1 = _m16(as1, ad1)
    op1, = edge_h8(src, dst, big1, ad1, m1, z80)
    big2, as2, ad2 = _tables_next(op1[0], op1[1], ex8, b1r, W2.astype(F32),
                                  Bs2, Bd2, pv)
    m2 = _m16(as2, ad2)
    op2, = edge_h8(src, dst, big2, ad2, m2, z80)
    big3, as3, ad3 = _tables3(op2[0], op2[1], ex8, b2r, W3.astype(F32),
                              A_s, A_d, pv)
    m3 = _m16(as3, ad3)
    op3, = edge_h1(src, dst, big3, ad3, m3, z80)
    s0, s1 = _final(op3[0], op3[1], ex1, b3e, b3o, W3e, W3o)
    return jnp.stack([s0[:N], s1[:N]], axis=-1)


# speedup vs baseline: 66.9178x; 1.0134x over previous
import functools

import jax
import jax.numpy as jnp
from jax import lax
from jax.experimental import pallas as pl
from jax.experimental.pallas import tpu as pltpu
from jax.experimental.pallas import tpu_sc as plsc

N = 10000
N_PAD = 10240
NC = 2
NS = 16
CHUNK = 128
CPT = 81
E_PAD = NC * NS * CPT * CHUNK
RPS = N_PAD // NS
NEGH = -5e29
F32 = jnp.float32


def _sc_mesh():
    return plsc.VectorSubcoreMesh(
        core_axis_name="c", subcore_axis_name="s", num_cores=NC, num_subcores=NS)


def _make_edge(heads8):
    def body(src_hbm, dst_hbm, big_hbm, adst_hbm, m16_hbm, z80_hbm,
             opart_hbm,
             src_v, dst_v, gs0, gd0, gs1, gd1, cm0, cm1, m16_v, acc_sp,
             sem_a, sem_b, sem_ca, sem_cb):
        c = lax.axis_index("c")
        s = lax.axis_index("s")
        wid = c * NS + s
        pltpu.sync_copy(z80_hbm.at[pl.ds(s * RPS, RPS)],
                        acc_sp.at[pl.ds(s * RPS, RPS)])
        pltpu.sync_copy(m16_hbm, m16_v)
        pltpu.sync_copy(src_hbm.at[wid], src_v)
        pltpu.sync_copy(dst_hbm.at[wid], dst_v)
        plsc.subcore_barrier()
        m16 = m16_v[...]
        iota16 = lax.broadcasted_iota(jnp.int32, (16,), 0)

        def issue(j, gs, gd, sem):
            pltpu.async_copy(big_hbm.at[src_v.at[j]], gs, sem)
            pltpu.async_copy(adst_hbm.at[dst_v.at[j]], gd, sem)

        def wait(gs, gd, sem):
            pltpu.make_async_copy(big_hbm.at[src_v.at[0]], gs, sem).wait()
            pltpu.make_async_copy(adst_hbm.at[dst_v.at[0]], gd, sem).wait()

        def compute(j, gs, gd, cm, sem_c):
            def edge_body(k, carry2):
                e = gs[k, pl.ds(0, 16)] + gd[k]
                e = jnp.where(e > 0, e, 0.2 * e)
                cm[k, pl.ds(64, 16)] = jnp.exp(e - m16)
                ksplat = jnp.full((16,), 0, jnp.int32) + k
                for v in range(4):
                    if heads8:
                        pat = 64 + 2 * v + jnp.where(iota16 >= 8, 1, 0)
                    else:
                        pat = 64 + iota16 * 0
                    av = plsc.load_gather(cm, [ksplat, pat])
                    cm[k, pl.ds(16 * v, 16)] = (
                        gs[k, pl.ds(16 + 16 * v, 16)] * av)
                return carry2

            lax.fori_loop(0, CHUNK, edge_body, 0, unroll=4)
            pltpu.async_copy(cm, acc_sp.at[dst_v.at[j]], sem_c, add=True)

        def drain(cm, sem_c):
            pltpu.make_async_copy(cm, acc_sp.at[dst_v.at[0]], sem_c).wait()

        issue(0, gs0, gd0, sem_a)

        def pair_body(t, carry):
            j0 = 2 * t
            issue(j0 + 1, gs1, gd1, sem_b)
            wait(gs0, gd0, sem_a)

            @pl.when(j0 >= 2)
            def _():
                drain(cm0, sem_ca)

            compute(j0, gs0, gd0, cm0, sem_ca)

            @pl.when(j0 + 2 < CPT)
            def _():
                issue(j0 + 2, gs0, gd0, sem_a)

            wait(gs1, gd1, sem_b)

            @pl.when(j0 >= 2)
            def _():
                drain(cm1, sem_cb)

            compute(j0 + 1, gs1, gd1, cm1, sem_cb)
            return carry

        lax.fori_loop(0, CPT // 2, pair_body, 0)
        wait(gs0, gd0, sem_a)
        drain(cm0, sem_ca)
        compute(CPT - 1, gs0, gd0, cm0, sem_ca)
        drain(cm0, sem_ca)
        drain(cm1, sem_cb)
        plsc.subcore_barrier()
        pltpu.sync_copy(acc_sp.at[pl.ds(s * RPS, RPS)],
                        opart_hbm.at[c, pl.ds(s * RPS, RPS)])

    return functools.partial(
        pl.kernel,
        out_type=[jax.ShapeDtypeStruct((NC, N_PAD, 80), F32)],
        mesh=_sc_mesh(),
        compiler_params=pltpu.CompilerParams(use_tc_tiling_on_sc=False,
                                             needs_layout_passes=False),
        scratch_types=[
            pltpu.VMEM((CPT, CHUNK), jnp.int32),
            pltpu.VMEM((CPT, CHUNK), jnp.int32),
            pltpu.VMEM((CHUNK, 80), F32),
            pltpu.VMEM((CHUNK, 16), F32),
            pltpu.VMEM((CHUNK, 80), F32),
            pltpu.VMEM((CHUNK, 16), F32),
            pltpu.VMEM((CHUNK, 80), F32),
            pltpu.VMEM((CHUNK, 80), F32),
            pltpu.VMEM((16,), F32),
            pltpu.VMEM_SHARED((N_PAD, 80), F32),
            pltpu.SemaphoreType.DMA,
            pltpu.SemaphoreType.DMA,
            pltpu.SemaphoreType.DMA,
            pltpu.SemaphoreType.DMA,
        ],
    )(body)


_BLK = 512
_NBLK = N_PAD // _BLK


def _tables1_body(x_ref, w_ref, bs_ref, bd_ref, pv_ref,
                  big_ref, as_ref, ad_ref):
    h = jnp.dot(x_ref[...], w_ref[...], preferred_element_type=F32)
    asrc = jnp.dot(h, bs_ref[...], preferred_element_type=F32) + pv_ref[...]
    big_ref[...] = jnp.concatenate([asrc, h], axis=1)
    as_ref[...] = asrc
    ad_ref[...] = jnp.dot(h, bd_ref[...], preferred_element_type=F32) + pv_ref[...]


def _tables_next_body(p0_ref, p1_ref, ex_ref, b_ref, w_ref, bs_ref, bd_ref,
                      pv_ref, big_ref, as_ref, ad_ref):
    p = p0_ref[...] + p1_ref[...]
    dexp = jnp.dot(p[:, 64:80], ex_ref[...], preferred_element_type=F32) + 1e-16
    xx = jax.nn.relu(p[:, 0:64] / dexp + b_ref[...])
    h = jnp.dot(xx, w_ref[...], preferred_element_type=F32)
    asrc = jnp.dot(h, bs_ref[...], preferred_element_type=F32) + pv_ref[...]
    big_ref[...] = jnp.concatenate([asrc, h], axis=1)
    as_ref[...] = asrc
    ad_ref[...] = jnp.dot(h, bd_ref[...], preferred_element_type=F32) + pv_ref[...]


def _tables3_body(p0_ref, p1_ref, ex_ref, b_ref, w_ref, as3_ref, ad3_ref,
                  pv_ref, big_ref, as_ref, ad_ref):
    p = p0_ref[...] + p1_ref[...]
    dexp = jnp.dot(p[:, 64:80], ex_ref[...], preferred_element_type=F32) + 1e-16
    xx = jax.nn.relu(p[:, 0:64] / dexp + b_ref[...])
    hw = jnp.dot(xx, w_ref[...], preferred_element_type=F32)
    asrc = jnp.dot(hw, as3_ref[...], preferred_element_type=F32) + pv_ref[...]
    big_ref[...] = jnp.concatenate([asrc, xx], axis=1)
    as_ref[...] = asrc
    ad_ref[...] = jnp.dot(hw, ad3_ref[...], preferred_element_type=F32) + pv_ref[...]


def _m16_body(as_ref, ad_ref, o_ref):
    m = jnp.max(as_ref[...], axis=0) + jnp.max(ad_ref[...], axis=0)
    m = jnp.where(m > 0, m, 0.2 * m)
    o_ref[...] = jnp.broadcast_to(m[None, :], (8, 16))


def _final_body(p0_ref, p1_ref, ex_ref, b3e_ref, b3o_ref, w3e_ref, w3o_ref,
                s0_ref, s1_ref):
    p = p0_ref[...] + p1_ref[...]
    dexp = jnp.dot(p[:, 64:80], ex_ref[...], preferred_element_type=F32) + 1e-16
    agg = p[:, 0:64] / dexp
    z0 = jax.nn.relu(jnp.dot(agg, w3e_ref[...], preferred_element_type=F32)
                     + b3e_ref[...])
    z1 = jax.nn.relu(jnp.dot(agg, w3o_ref[...], preferred_element_type=F32)
                     + b3o_ref[...])
    m = jnp.maximum(z0, z1)
    e0 = jnp.exp(z0 - m)
    e1 = jnp.exp(z1 - m)
    t = e0 + e1
    s0_ref[...] = e0 / t
    s1_ref[...] = e1 / t


def _row_spec(cols):
    return pl.BlockSpec((_BLK, cols), lambda i: (i, 0))


def _full_spec(shape):
    return pl.BlockSpec(shape, lambda i: tuple(0 for _ in shape))


_TBL_OUT = [jax.ShapeDtypeStruct((N_PAD, 80), F32),
            jax.ShapeDtypeStruct((N_PAD, 16), F32),
            jax.ShapeDtypeStruct((N_PAD, 16), F32)]
_TBL_OUT_SPECS = [_row_spec(80), _row_spec(16), _row_spec(16)]


def _tables1(xp, W1, Bs, Bd, pv):
    return pl.pallas_call(
        _tables1_body,
        grid=(_NBLK,),
        in_specs=[_row_spec(128), _full_spec((128, 64)), _full_spec((64, 16)),
                  _full_spec((64, 16)), _full_spec((1, 16))],
        out_specs=_TBL_OUT_SPECS,
        out_shape=_TBL_OUT,
    )(xp, W1, Bs, Bd, pv)


def _tables_next(p0, p1, ex, b, W, Bs, Bd, pv):
    return pl.pallas_call(
        _tables_next_body,
        grid=(_NBLK,),
        in_specs=[_row_spec(80), _row_spec(80), _full_spec((16, 64)),
                  _full_spec((1, 64)), _full_spec((64, 64)),
                  _full_spec((64, 16)), _full_spec((64, 16)),
                  _full_spec((1, 16))],
        out_specs=_TBL_OUT_SPECS,
        out_shape=_TBL_OUT,
    )(p0, p1, ex, b, W, Bs, Bd, pv)


def _tables3(p0, p1, ex, b, W3, A_s, A_d, pv):
    return pl.pallas_call(
        _tables3_body,
        grid=(_NBLK,),
        in_specs=[_row_spec(80), _row_spec(80), _full_spec((16, 64)),
                  _full_spec((1, 64)), _full_spec((64, 242)),
                  _full_spec((242, 16)), _full_spec((242, 16)),
                  _full_spec((1, 16))],
        out_specs=_TBL_OUT_SPECS,
        out_shape=_TBL_OUT,
    )(p0, p1, ex, b, W3, A_s, A_d, pv)


def _m16(asrc, adst):
    out = pl.pallas_call(
        _m16_body,
        out_shape=jax.ShapeDtypeStruct((8, 16), F32),
    )(asrc, adst)
    return out[0]


def _final(p0, p1, ex, b3e, b3o, W3e, W3o):
    return pl.pallas_call(
        _final_body,
        grid=(_NBLK,),
        in_specs=[_row_spec(80), _row_spec(80), _full_spec((16, 64)),
                  _full_spec((1, 121)), _full_spec((1, 121)),
                  _full_spec((64, 121)), _full_spec((64, 121))],
        out_specs=[_row_spec(121), _row_spec(121)],
        out_shape=[jax.ShapeDtypeStruct((N_PAD, 121), F32),
                   jax.ShapeDtypeStruct((N_PAD, 121), F32)],
    )(p0, p1, ex, b3e, b3o, W3e, W3o)


def _blockdiag(a):
    eye8 = jnp.eye(8, dtype=F32)
    B = (a.astype(F32)[:, :, None] * eye8[:, None, :]).reshape(64, 8)
    return jnp.pad(B, ((0, 0), (0, 8)))


def kernel(x, edge_index, W1, a_src1, a_dst1, b1, W2, a_src2, a_dst2, b2,
           W3, a_src3, a_dst3, b3):
    x = x.astype(F32)
    loop = jnp.arange(N, dtype=jnp.int32)
    src = jnp.concatenate([edge_index[0].astype(jnp.int32), loop])
    dst = jnp.concatenate([edge_index[1].astype(jnp.int32), loop])
    pad_e = E_PAD - src.shape[0]
    src = jnp.concatenate([src, jnp.full((pad_e,), N, jnp.int32)])
    dst = jnp.concatenate([dst, jnp.full((pad_e,), N, jnp.int32)])
    src = src.reshape(NC * NS, CPT, CHUNK)
    dst = dst.reshape(NC * NS, CPT, CHUNK)

    xp = jnp.pad(x, ((0, N_PAD - N), (0, 0)))
    Bs1, Bd1 = _blockdiag(a_src1), _blockdiag(a_dst1)
    Bs2, Bd2 = _blockdiag(a_src2), _blockdiag(a_dst2)
    A_s = jnp.pad(jnp.tile(a_src3.astype(F32).reshape(242, 1), (1, 8)),
                  ((0, 0), (0, 8)))
    A_d = jnp.pad(jnp.tile(a_dst3.astype(F32).reshape(242, 1), (1, 8)),
                  ((0, 0), (0, 8)))
    pv = jnp.concatenate([jnp.zeros((8,), F32),
                          jnp.full((8,), NEGH, F32)]).reshape(1, 16)
    ex8 = jnp.pad(jnp.kron(jnp.eye(8, dtype=F32), jnp.ones((1, 8), F32)),
                  ((0, 8), (0, 0)))
    ex1 = jnp.zeros((16, 64), F32).at[0, :].set(1.0)
    z80 = jnp.zeros((N_PAD, 80), F32)
    W3e = W3.astype(F32)[:, 0::2]
    W3o = W3.astype(F32)[:, 1::2]
    b3e = b3.astype(F32)[0::2].reshape(1, 121)
    b3o = b3.astype(F32)[1::2].reshape(1, 121)
    b1r = b1.astype(F32).reshape(1, 64)
    b2r = b2.astype(F32).reshape(1, 64)

    edge_h8 = _make_edge(True)
    edge_h1 = _make_edge(False)

    big1, as1, ad1 = _tables1(xp, W1.astype(F32), Bs1, Bd1, pv)
    m1 = _m16(as1, ad1)
    op1, = edge_h8(src, dst, big1, ad1, m1, z80)
    big2, as2, ad2 = _tables_next(op1[0], op1[1], ex8, b1r, W2.astype(F32),
                                  Bs2, Bd2, pv)
    m2 = _m16(as2, ad2)
    op2, = edge_h8(src, dst, big2, ad2, m2, z80)
    big3, as3, ad3 = _tables3(op2[0], op2[1], ex8, b2r, W3.astype(F32),
                              A_s, A_d, pv)
    m3 = _m16(as3, ad3)
    op3, = edge_h1(src, dst, big3, ad3, m3, z80)
    s0, s1 = _final(op3[0], op3[1], ex1, b3e, b3o, W3e, W3o)
    return jnp.stack([s0[:N], s1[:N]], axis=-1)
